# baseline (speedup 1.0000x reference)
import functools

import jax
import jax.numpy as jnp
from jax import lax
from jax.experimental import pallas as pl
from jax.experimental.pallas import tpu as pltpu
from jax.experimental.pallas import tpu_sc as plsc

N = 10000
E = 320000
T = 4
D = 128
H = 128
O = 128
HH = H // 2

NP = 10240

NC = 2
NS = 16
EB = 80
CH = 32
NCH = 8
RPS = NCH * CH
EP = NS * RPS * EB
NPS = NP // NS

BR = 2000
NBLK = N // BR
LBR = 1000
NLBLK = N // LBR

F32 = jnp.float32

_sc_mesh = plsc.VectorSubcoreMesh(
    core_axis_name="c", subcore_axis_name="s", num_cores=NC, num_subcores=NS
)
_sc_params = pltpu.CompilerParams(use_tc_tiling_on_sc=False)


@functools.partial(
    pl.kernel,
    out_type=jax.ShapeDtypeStruct((T, NP, 16), F32),
    mesh=_sc_mesh,
    compiler_params=_sc_params,
    scratch_types=[
        pltpu.VMEM((CH, EB), jnp.int32),
        pltpu.VMEM((EB, 16), F32),
        pltpu.VMEM_SHARED((NP, 16), F32),
        pltpu.SemaphoreType.DMA,
    ],
)
def _deg_kernel(dst_hbm, ones_hbm, zeros_hbm, deg_out, dstbuf, valbuf, accd,
                dsem):
    cid = lax.axis_index("c")
    sid = lax.axis_index("s")
    pltpu.sync_copy(ones_hbm, valbuf)

    def one_t(t):
        pltpu.sync_copy(zeros_hbm.at[pl.ds(sid * NPS, NPS)],
                        accd.at[pl.ds(sid * NPS, NPS)])
        plsc.subcore_barrier()

        @pl.loop(0, NCH)
        def _(k):
            pltpu.sync_copy(dst_hbm.at[t].at[sid].at[k], dstbuf)

            @pl.loop(0, CH // 8)
            def _(g):
                for u in range(8):
                    pltpu.make_async_copy(
                        valbuf, accd.at[dstbuf.at[g * 8 + u]],
                        dsem).start(add=True)
                for u in range(8):
                    pltpu.make_async_copy(
                        valbuf, accd.at[dstbuf.at[g * 8 + u]],
                        dsem).wait()

        plsc.subcore_barrier()
        pltpu.sync_copy(accd.at[pl.ds(sid * NPS, NPS)],
                        deg_out.at[t].at[pl.ds(sid * NPS, NPS)])
        plsc.subcore_barrier()

    for c in range(NC):
        @pl.when(cid == c)
        def _(c=c):
            for t in range(2 * c, 2 * c + 2):
                one_t(t)


G = 4
NG = CH // G


@functools.partial(
    pl.kernel,
    out_type=jax.ShapeDtypeStruct((T, NP, H), F32),
    mesh=_sc_mesh,
    compiler_params=_sc_params,
    scratch_types=[
        pltpu.VMEM((CH, EB), jnp.int32),
        pltpu.VMEM((CH, EB), jnp.int32),
        pltpu.VMEM((G, EB, H), F32),
        pltpu.VMEM_SHARED((NP, H), F32),
        pltpu.SemaphoreType.DMA((G,)),
        pltpu.SemaphoreType.DMA((G,)),
    ],
)
def _agg_kernel(tab_hbm, src_hbm, dst_hbm, out_hbm, srcbuf, dstbuf, rows, acc,
                gsem, ssem):
    cid = lax.axis_index("c")
    sid = lax.axis_index("s")

    def one_pass(t):
        pltpu.sync_copy(tab_hbm.at[t].at[pl.ds(sid * NPS, NPS)],
                        acc.at[pl.ds(sid * NPS, NPS)])
        plsc.subcore_barrier()

        def fire_g(j, u):
            pltpu.make_async_copy(tab_hbm.at[t].at[srcbuf.at[j]],
                                  rows.at[u], gsem.at[u]).start()

        def wait_g(j, u):
            pltpu.make_async_copy(tab_hbm.at[t].at[srcbuf.at[j]],
                                  rows.at[u], gsem.at[u]).wait()

        def fire_s(j, u):
            pltpu.make_async_copy(rows.at[u], acc.at[dstbuf.at[j]],
                                  ssem.at[u]).start(add=True)

        def wait_s(j, u):
            pltpu.make_async_copy(rows.at[u], acc.at[dstbuf.at[j]],
                                  ssem.at[u]).wait()

        @pl.loop(0, NCH)
        def _(k):
            pltpu.sync_copy(src_hbm.at[t].at[sid].at[k], srcbuf)
            pltpu.sync_copy(dst_hbm.at[t].at[sid].at[k], dstbuf)
            for u in range(G):
                fire_g(u, u)

            @pl.loop(0, CH // G - 1)
            def _(g):
                for u in range(G):
                    j = g * G + u
                    wait_g(j, u)
                    fire_s(j, u)
                    wait_s(j, u)
                    fire_g(j + G, u)

            for u in range(G):
                j = CH - G + u
                wait_g(j, u)
                fire_s(j, u)
            for u in range(G):
                wait_s(CH - G + u, u)

        plsc.subcore_barrier()
        pltpu.sync_copy(acc.at[pl.ds(sid * NPS, NPS)],
                        out_hbm.at[t].at[pl.ds(sid * NPS, NPS)])
        plsc.subcore_barrier()

    for c in range(NC):
        @pl.when(cid == c)
        def _(c=c):
            for t in range(2 * c, 2 * c + 2):
                one_pass(t)


def _mm_body(x_ref, deg_ref, w_ref, o_ref):
    o_ref[0] = jnp.dot(x_ref[0], w_ref[...],
                       preferred_element_type=F32) * _dinv_of(deg_ref[0])


def _matmul_xw(x, deg, w):
    return pl.pallas_call(
        _mm_body,
        grid=(T, NBLK),
        in_specs=[
            pl.BlockSpec((1, BR, D), lambda t, i: (t, i, 0)),
            pl.BlockSpec((1, BR, 16), lambda t, i: (t, i, 0)),
            pl.BlockSpec((D, H), lambda t, i: (0, 0)),
        ],
        out_specs=pl.BlockSpec((1, BR, H), lambda t, i: (t, i, 0)),
        out_shape=jax.ShapeDtypeStruct((T, NP, H), F32),
    )(x, deg, w)


def _dinv_of(deg_blk):
    return jax.lax.rsqrt(deg_blk[:, 0:1] + 1.0)


def _mid_body(agg_ref, deg_ref, b_ref, w_ref, o_ref):
    dinv = _dinv_of(deg_ref[0])
    z = jnp.maximum(agg_ref[0] * dinv + b_ref[...], 0.0)
    o_ref[0] = jnp.dot(z, w_ref[...], preferred_element_type=F32) * dinv


def _mid_layer(agg, deg, b1, w2):
    return pl.pallas_call(
        _mid_body,
        grid=(T, NBLK),
        in_specs=[
            pl.BlockSpec((1, BR, H), lambda t, i: (t, i, 0)),
            pl.BlockSpec((1, BR, 16), lambda t, i: (t, i, 0)),
            pl.BlockSpec((1, H), lambda t, i: (0, 0)),
            pl.BlockSpec((H, H), lambda t, i: (0, 0)),
        ],
        out_specs=pl.BlockSpec((1, BR, H), lambda t, i: (t, i, 0)),
        out_shape=jax.ShapeDtypeStruct((T, NP, H), F32),
    )(agg, deg, b1, w2)


def _lstm_body(agg_ref, deg_ref, b2_ref, wih_ref, whh_ref, bsum_ref,
               wout_ref, bout_ref, out_ref, hs_ref, cs_ref):
    h = jnp.zeros((LBR, H), F32)
    c = jnp.zeros((LBR, H), F32)
    for t in range(T):
        dinv = _dinv_of(deg_ref[t])
        z = jnp.maximum(agg_ref[t] * dinv + b2_ref[...], 0.0)
        gates = (jnp.dot(z, wih_ref[...], preferred_element_type=F32)
                 + jnp.dot(h, whh_ref[...], preferred_element_type=F32)
                 + bsum_ref[...])
        gi = jax.nn.sigmoid(gates[:, 0:H])
        gf = jax.nn.sigmoid(gates[:, H:2 * H])
        gg = jnp.tanh(gates[:, 2 * H:3 * H])
        go = jax.nn.sigmoid(gates[:, 3 * H:4 * H])
        c = gf * c + gi * gg
        h = go * jnp.tanh(c)
    out_ref[...] = jnp.dot(h, wout_ref[...],
                           preferred_element_type=F32) + bout_ref[...]
    hs_ref[0] = h
    cs_ref[0] = c


def _lstm_out(agg, deg, b2, wihT, whhT, bsum, woutT, bout):
    return pl.pallas_call(
        _lstm_body,
        grid=(NLBLK,),
        in_specs=[
            pl.BlockSpec((T, LBR, H), lambda i: (0, i, 0)),
            pl.BlockSpec((T, LBR, 16), lambda i: (0, i, 0)),
            pl.BlockSpec((1, H), lambda i: (0, 0)),
            pl.BlockSpec((H, 4 * H), lambda i: (0, 0)),
            pl.BlockSpec((H, 4 * H), lambda i: (0, 0)),
            pl.BlockSpec((1, 4 * H), lambda i: (0, 0)),
            pl.BlockSpec((H, O), lambda i: (0, 0)),
            pl.BlockSpec((1, O), lambda i: (0, 0)),
        ],
        out_specs=[
            pl.BlockSpec((LBR, O), lambda i: (i, 0)),
            pl.BlockSpec((1, LBR, H), lambda i: (0, i, 0)),
            pl.BlockSpec((1, LBR, H), lambda i: (0, i, 0)),
        ],
        out_shape=[
            jax.ShapeDtypeStruct((N, O), F32),
            jax.ShapeDtypeStruct((1, N, H), F32),
            jax.ShapeDtypeStruct((1, N, H), F32),
        ],
    )(agg, deg, b2, wihT, whhT, bsum, woutT, bout)


def kernel(x_sequence, edge_index_sequence, W_gcn1, b_gcn1, W_gcn2, b_gcn2,
           W_ih, W_hh, b_ih, b_hh, W_out, b_out):
    pad = jnp.full((T, 2, EP - E), N, jnp.int32)
    eidx = jnp.concatenate([edge_index_sequence, pad], axis=2)
    src_all = eidx[:, 0, :].reshape(T, NS, NCH, CH, EB)
    dst_all = eidx[:, 1, :].reshape(T, NS, NCH, CH, EB)
    ones_rows = jnp.ones((EB, 16), F32)
    zeros_n16 = jnp.zeros((NP, 16), F32)

    deg = _deg_kernel(dst_all, ones_rows, zeros_n16)
    h1s = _matmul_xw(x_sequence, deg, W_gcn1)
    agg1 = _agg_kernel(h1s, src_all, dst_all)
    h2s = _mid_layer(agg1, deg, b_gcn1.reshape(1, H), W_gcn2)
    agg2 = _agg_kernel(h2s, src_all, dst_all)
    return _lstm_out(
        agg2, deg, b_gcn2.reshape(1, H), W_ih.T, W_hh.T,
        (b_ih + b_hh).reshape(1, 4 * H), W_out.T, b_out.reshape(1, O))


# speedup vs baseline: 1.1298x; 1.1298x over previous
import functools

import jax
import jax.numpy as jnp
from jax import lax
from jax.experimental import pallas as pl
from jax.experimental.pallas import tpu as pltpu
from jax.experimental.pallas import tpu_sc as plsc

N = 10000
E = 320000
T = 4
D = 128
H = 128
O = 128
HH = H // 2

NP = 10240

NC = 2
NS = 16
EB = 80
CH = 32
NCH = 8
RPS = NCH * CH
EP = NS * RPS * EB
NPS = NP // NS

BR = 2048
NBLK = NP // BR
LBR = 1024
NLBLK = NP // LBR

F32 = jnp.float32

_sc_mesh = plsc.VectorSubcoreMesh(
    core_axis_name="c", subcore_axis_name="s", num_cores=NC, num_subcores=NS
)
_sc_params = pltpu.CompilerParams(use_tc_tiling_on_sc=False)


@functools.partial(
    pl.kernel,
    out_type=jax.ShapeDtypeStruct((T, NP, 16), F32),
    mesh=_sc_mesh,
    compiler_params=_sc_params,
    scratch_types=[
        pltpu.VMEM((CH, EB), jnp.int32),
        pltpu.VMEM((EB, 16), F32),
        pltpu.VMEM_SHARED((NP, 16), F32),
        pltpu.SemaphoreType.DMA,
    ],
)
def _deg_kernel(dst_hbm, ones_hbm, zeros_hbm, deg_out, dstbuf, valbuf, accd,
                dsem):
    cid = lax.axis_index("c")
    sid = lax.axis_index("s")
    pltpu.sync_copy(ones_hbm, valbuf)

    def one_t(t):
        pltpu.sync_copy(zeros_hbm.at[pl.ds(sid * NPS, NPS)],
                        accd.at[pl.ds(sid * NPS, NPS)])
        plsc.subcore_barrier()

        @pl.loop(0, NCH)
        def _(k):
            pltpu.sync_copy(dst_hbm.at[t].at[sid].at[k], dstbuf)

            @pl.loop(0, CH // 8)
            def _(g):
                for u in range(8):
                    pltpu.make_async_copy(
                        valbuf, accd.at[dstbuf.at[g * 8 + u]],
                        dsem).start(add=True)
                for u in range(8):
                    pltpu.make_async_copy(
                        valbuf, accd.at[dstbuf.at[g * 8 + u]],
                        dsem).wait()

        plsc.subcore_barrier()
        pltpu.sync_copy(accd.at[pl.ds(sid * NPS, NPS)],
                        deg_out.at[t].at[pl.ds(sid * NPS, NPS)])
        plsc.subcore_barrier()

    for c in range(NC):
        @pl.when(cid == c)
        def _(c=c):
            for t in range(2 * c, 2 * c + 2):
                one_t(t)


G = 4
NG = CH // G


@functools.partial(
    pl.kernel,
    out_type=jax.ShapeDtypeStruct((T, NP, H), F32),
    mesh=_sc_mesh,
    compiler_params=_sc_params,
    scratch_types=[
        pltpu.VMEM((CH, EB), jnp.int32),
        pltpu.VMEM((CH, EB), jnp.int32),
        pltpu.VMEM((G, EB, H), F32),
        pltpu.VMEM_SHARED((NP, H), F32),
        pltpu.SemaphoreType.DMA((G,)),
        pltpu.SemaphoreType.DMA((G,)),
    ],
)
def _agg_kernel(tab_hbm, src_hbm, dst_hbm, out_hbm, srcbuf, dstbuf, rows, acc,
                gsem, ssem):
    cid = lax.axis_index("c")
    sid = lax.axis_index("s")

    def one_pass(t):
        pltpu.sync_copy(tab_hbm.at[t].at[pl.ds(sid * NPS, NPS)],
                        acc.at[pl.ds(sid * NPS, NPS)])
        plsc.subcore_barrier()

        def fire_g(j, u):
            pltpu.make_async_copy(tab_hbm.at[t].at[srcbuf.at[j]],
                                  rows.at[u], gsem.at[u]).start()

        def wait_g(j, u):
            pltpu.make_async_copy(tab_hbm.at[t].at[srcbuf.at[j]],
                                  rows.at[u], gsem.at[u]).wait()

        def fire_s(j, u):
            pltpu.make_async_copy(rows.at[u], acc.at[dstbuf.at[j]],
                                  ssem.at[u]).start(add=True)

        def wait_s(j, u):
            pltpu.make_async_copy(rows.at[u], acc.at[dstbuf.at[j]],
                                  ssem.at[u]).wait()

        @pl.loop(0, NCH)
        def _(k):
            pltpu.sync_copy(src_hbm.at[t].at[sid].at[k], srcbuf)
            pltpu.sync_copy(dst_hbm.at[t].at[sid].at[k], dstbuf)
            for u in range(G):
                fire_g(u, u)

            @pl.loop(0, CH // G - 1)
            def _(g):
                for u in range(G):
                    j = g * G + u
                    wait_g(j, u)
                    fire_s(j, u)
                    wait_s(j, u)
                    fire_g(j + G, u)

            for u in range(G):
                j = CH - G + u
                wait_g(j, u)
                fire_s(j, u)
            for u in range(G):
                wait_s(CH - G + u, u)

        plsc.subcore_barrier()
        pltpu.sync_copy(acc.at[pl.ds(sid * NPS, NPS)],
                        out_hbm.at[t].at[pl.ds(sid * NPS, NPS)])
        plsc.subcore_barrier()

    for c in range(NC):
        @pl.when(cid == c)
        def _(c=c):
            for t in range(2 * c, 2 * c + 2):
                one_pass(t)


def _mm_body(x_ref, deg_ref, w_ref, o_ref):
    o_ref[0] = jnp.dot(x_ref[0], w_ref[...],
                       preferred_element_type=F32) * _dinv_of(deg_ref[0])


def _matmul_xw(x, deg, w):
    return pl.pallas_call(
        _mm_body,
        grid=(T, NBLK),
        in_specs=[
            pl.BlockSpec((1, BR, D), lambda t, i: (t, i, 0)),
            pl.BlockSpec((1, BR, 16), lambda t, i: (t, i, 0)),
            pl.BlockSpec((D, H), lambda t, i: (0, 0)),
        ],
        out_specs=pl.BlockSpec((1, BR, H), lambda t, i: (t, i, 0)),
        out_shape=jax.ShapeDtypeStruct((T, NP, H), F32),
    )(x, deg, w)


def _dinv_of(deg_blk):
    return jax.lax.rsqrt(deg_blk[:, 0:1] + 1.0)


def _mid_body(agg_ref, deg_ref, b_ref, w_ref, o_ref):
    dinv = _dinv_of(deg_ref[0])
    z = jnp.maximum(agg_ref[0] * dinv + b_ref[...], 0.0)
    o_ref[0] = jnp.dot(z, w_ref[...], preferred_element_type=F32) * dinv


def _mid_layer(agg, deg, b1, w2):
    return pl.pallas_call(
        _mid_body,
        grid=(T, NBLK),
        in_specs=[
            pl.BlockSpec((1, BR, H), lambda t, i: (t, i, 0)),
            pl.BlockSpec((1, BR, 16), lambda t, i: (t, i, 0)),
            pl.BlockSpec((1, H), lambda t, i: (0, 0)),
            pl.BlockSpec((H, H), lambda t, i: (0, 0)),
        ],
        out_specs=pl.BlockSpec((1, BR, H), lambda t, i: (t, i, 0)),
        out_shape=jax.ShapeDtypeStruct((T, NP, H), F32),
    )(agg, deg, b1, w2)


def _lstm_body(agg_ref, deg_ref, b2_ref, wih_ref, whh_ref, bsum_ref,
               wout_ref, bout_ref, out_ref, hs_ref, cs_ref):
    h = jnp.zeros((LBR, H), F32)
    c = jnp.zeros((LBR, H), F32)
    for t in range(T):
        dinv = _dinv_of(deg_ref[t])
        z = jnp.maximum(agg_ref[t] * dinv + b2_ref[...], 0.0)
        gates = (jnp.dot(z, wih_ref[...], preferred_element_type=F32)
                 + jnp.dot(h, whh_ref[...], preferred_element_type=F32)
                 + bsum_ref[...])
        gi = jax.nn.sigmoid(gates[:, 0:H])
        gf = jax.nn.sigmoid(gates[:, H:2 * H])
        gg = jnp.tanh(gates[:, 2 * H:3 * H])
        go = jax.nn.sigmoid(gates[:, 3 * H:4 * H])
        c = gf * c + gi * gg
        h = go * jnp.tanh(c)
    out_ref[...] = jnp.dot(h, wout_ref[...],
                           preferred_element_type=F32) + bout_ref[...]
    hs_ref[0] = h
    cs_ref[0] = c


def _lstm_out(agg, deg, b2, wihT, whhT, bsum, woutT, bout):
    return pl.pallas_call(
        _lstm_body,
        grid=(NLBLK,),
        in_specs=[
            pl.BlockSpec((T, LBR, H), lambda i: (0, i, 0)),
            pl.BlockSpec((T, LBR, 16), lambda i: (0, i, 0)),
            pl.BlockSpec((1, H), lambda i: (0, 0)),
            pl.BlockSpec((H, 4 * H), lambda i: (0, 0)),
            pl.BlockSpec((H, 4 * H), lambda i: (0, 0)),
            pl.BlockSpec((1, 4 * H), lambda i: (0, 0)),
            pl.BlockSpec((H, O), lambda i: (0, 0)),
            pl.BlockSpec((1, O), lambda i: (0, 0)),
        ],
        out_specs=[
            pl.BlockSpec((LBR, O), lambda i: (i, 0)),
            pl.BlockSpec((1, LBR, H), lambda i: (0, i, 0)),
            pl.BlockSpec((1, LBR, H), lambda i: (0, i, 0)),
        ],
        out_shape=[
            jax.ShapeDtypeStruct((NP, O), F32),
            jax.ShapeDtypeStruct((1, NP, H), F32),
            jax.ShapeDtypeStruct((1, NP, H), F32),
        ],
    )(agg, deg, b2, wihT, whhT, bsum, woutT, bout)


def kernel(x_sequence, edge_index_sequence, W_gcn1, b_gcn1, W_gcn2, b_gcn2,
           W_ih, W_hh, b_ih, b_hh, W_out, b_out):
    pad = jnp.full((T, 2, EP - E), N, jnp.int32)
    eidx = jnp.concatenate([edge_index_sequence, pad], axis=2)
    src_all = eidx[:, 0, :].reshape(T, NS, NCH, CH, EB)
    dst_all = eidx[:, 1, :].reshape(T, NS, NCH, CH, EB)
    ones_rows = jnp.ones((EB, 16), F32)
    zeros_n16 = jnp.zeros((NP, 16), F32)
    x_pad = jnp.pad(x_sequence, ((0, 0), (0, NP - N), (0, 0)))

    deg = _deg_kernel(dst_all, ones_rows, zeros_n16)
    h1s = _matmul_xw(x_pad, deg, W_gcn1)
    agg1 = _agg_kernel(h1s, src_all, dst_all)
    h2s = _mid_layer(agg1, deg, b_gcn1.reshape(1, H), W_gcn2)
    agg2 = _agg_kernel(h2s, src_all, dst_all)
    out, hs, cs = _lstm_out(
        agg2, deg, b_gcn2.reshape(1, H), W_ih.T, W_hh.T,
        (b_ih + b_hh).reshape(1, 4 * H), W_out.T, b_out.reshape(1, O))
    return (out[:N], hs[:, :N], cs[:, :N])
